# baseline (speedup 1.0000x reference)
import functools

import jax
import jax.numpy as jnp
from jax import lax
from jax.experimental import pallas as pl
from jax.experimental.pallas import tpu as pltpu
from jax.experimental.pallas import tpu_sc as plsc

NUM_CORES = 2
NUM_SUBCORES = 16
NUM_WORKERS = NUM_CORES * NUM_SUBCORES

K = 8

DEPAD_ROWS = 160


def _depad_sc(table):
    v, d = table.shape
    n_chunks = v // DEPAD_ROWS
    per_w = -(-n_chunks // NUM_WORKERS)
    half = DEPAD_ROWS // 2

    mesh = plsc.VectorSubcoreMesh(
        core_axis_name="c", subcore_axis_name="s",
        num_cores=NUM_CORES, num_subcores=NUM_SUBCORES)

    @functools.partial(
        pl.kernel,
        mesh=mesh,
        out_type=jax.ShapeDtypeStruct((v // 2, 2 * d), jnp.float32),
        scratch_types=[
            pltpu.VMEM((DEPAD_ROWS, d), jnp.float32),
            pltpu.VMEM((half, 2 * d), jnp.float32),
        ],
    )
    def body(table_hbm, out_hbm, inb, outb):
        wid = lax.axis_index("s") * NUM_CORES + lax.axis_index("c")

        @pl.loop(0, per_w)
        def _chunks(i):
            c = i * NUM_WORKERS + wid

            @pl.when(c < n_chunks)
            def _do():
                pltpu.sync_copy(
                    table_hbm.at[pl.ds(c * DEPAD_ROWS, DEPAD_ROWS)], inb)

                @pl.loop(0, half)
                def _rows(r2):
                    for k in range(d // 16):
                        outb[r2, pl.ds(16 * k, 16)] = (
                            inb[2 * r2, pl.ds(16 * k, 16)])
                        outb[r2, pl.ds(d + 16 * k, 16)] = (
                            inb[2 * r2 + 1, pl.ds(16 * k, 16)])

                pltpu.sync_copy(outb, out_hbm.at[pl.ds(c * half, half)])

    return body(table)


def _gather_sc(table, ids):
    n_tok, n_per = ids.shape
    d = table.shape[1]
    tok_per_w = n_tok // NUM_WORKERS
    n_steps = tok_per_w // K
    assert n_steps % 2 == 0 and n_per <= 128

    mesh = plsc.VectorSubcoreMesh(
        core_axis_name="c", subcore_axis_name="s",
        num_cores=NUM_CORES, num_subcores=NUM_SUBCORES)

    @functools.partial(
        pl.kernel,
        mesh=mesh,
        out_type=jax.ShapeDtypeStruct((n_tok, n_per, d), jnp.float32),
        compiler_params=pltpu.CompilerParams(use_tc_tiling_on_sc=False),
        scratch_types=[
            pltpu.VMEM((tok_per_w, n_per), jnp.int32),
            pltpu.VMEM((2, K, n_per, d), jnp.float32),
            pltpu.SemaphoreType.DMA,
            pltpu.SemaphoreType.DMA,
            pltpu.SemaphoreType.DMA,
            pltpu.SemaphoreType.DMA,
        ],
    )
    def body(table_hbm, ids_hbm, out_hbm, idx_v, rows_v, g0, g1, w0, w1):
        wid = lax.axis_index("s") * NUM_CORES + lax.axis_index("c")
        tok_base = wid * tok_per_w
        gsem = (g0, g1)
        wsem = (w0, w1)
        pltpu.sync_copy(ids_hbm.at[pl.ds(tok_base, tok_per_w)], idx_v)

        def fire_gathers(s, b):
            for j in range(K):
                pltpu.async_copy(
                    table_hbm.at[idx_v.at[s * K + j]],
                    rows_v.at[b, j],
                    gsem[b])

        def wait_gathers(b):
            for j in range(K):
                pltpu.make_async_copy(
                    table_hbm.at[pl.ds(0, n_per)], rows_v.at[b, j],
                    gsem[b]).wait()

        def fire_write(s, b):
            pltpu.async_copy(
                rows_v.at[b],
                out_hbm.at[pl.ds(tok_base + s * K, K)],
                wsem[b])

        def wait_write(b):
            pltpu.make_async_copy(
                rows_v.at[b], out_hbm.at[pl.ds(tok_base, K)], wsem[b]).wait()

        fire_gathers(0, 0)

        @pl.loop(0, n_steps, step=2)
        def _steps(t):
            for b in range(2):
                s = t + b
                b2 = 1 - b

                @pl.when(s + 1 < n_steps)
                def _prefetch():
                    @pl.when(s >= 1)
                    def _drain():
                        wait_write(b2)
                    fire_gathers(s + 1, b2)

                wait_gathers(b)
                fire_write(s, b)

        wait_write(0)
        wait_write(1)

    return body(table, ids)


def kernel(token_ids, Embedding_Matrix):
    v, d = Embedding_Matrix.shape
    dense = _depad_sc(Embedding_Matrix)
    table_lin = dense.reshape(v, d)
    return _gather_sc(table_lin, token_ids.astype(jnp.int32))


# speedup vs baseline: 1.4560x; 1.4560x over previous
import functools

import jax
import jax.numpy as jnp
from jax import lax
from jax.experimental import pallas as pl
from jax.experimental.pallas import tpu as pltpu
from jax.experimental.pallas import tpu_sc as plsc

NUM_CORES = 2
NUM_SUBCORES = 16
NUM_WORKERS = NUM_CORES * NUM_SUBCORES

K = 8


def _gather_sc(table, ids):
    n_tok, n_per = ids.shape
    d = table.shape[1]
    tok_per_w = n_tok // NUM_WORKERS
    n_steps = tok_per_w // K
    assert n_steps % 2 == 0 and n_per <= 128

    mesh = plsc.VectorSubcoreMesh(
        core_axis_name="c", subcore_axis_name="s",
        num_cores=NUM_CORES, num_subcores=NUM_SUBCORES)

    @functools.partial(
        pl.kernel,
        mesh=mesh,
        out_type=jax.ShapeDtypeStruct((n_tok, n_per, d), jnp.float32),
        compiler_params=pltpu.CompilerParams(use_tc_tiling_on_sc=False),
        scratch_types=[
            pltpu.VMEM((tok_per_w, n_per), jnp.int32),
            pltpu.VMEM((2, K, n_per, d), jnp.float32),
            pltpu.SemaphoreType.DMA,
            pltpu.SemaphoreType.DMA,
            pltpu.SemaphoreType.DMA,
            pltpu.SemaphoreType.DMA,
        ],
    )
    def body(table_hbm, ids_hbm, out_hbm, idx_v, rows_v, g0, g1, w0, w1):
        wid = lax.axis_index("s") * NUM_CORES + lax.axis_index("c")
        tok_base = wid * tok_per_w
        gsem = (g0, g1)
        wsem = (w0, w1)
        pltpu.sync_copy(ids_hbm.at[pl.ds(tok_base, tok_per_w)], idx_v)

        def fire_gathers(s, b):
            for j in range(K):
                pltpu.async_copy(
                    table_hbm.at[idx_v.at[s * K + j]],
                    rows_v.at[b, j],
                    gsem[b])

        def wait_gathers(b):
            for j in range(K):
                pltpu.make_async_copy(
                    table_hbm.at[pl.ds(0, n_per)], rows_v.at[b, j],
                    gsem[b]).wait()

        def fire_write(s, b):
            pltpu.async_copy(
                rows_v.at[b],
                out_hbm.at[pl.ds(tok_base + s * K, K)],
                wsem[b])

        def wait_write(b):
            pltpu.make_async_copy(
                rows_v.at[b], out_hbm.at[pl.ds(tok_base, K)], wsem[b]).wait()

        fire_gathers(0, 0)

        @pl.loop(0, n_steps, step=2)
        def _steps(t):
            for b in range(2):
                s = t + b
                b2 = 1 - b

                @pl.when(s + 1 < n_steps)
                def _prefetch():
                    @pl.when(s >= 1)
                    def _drain():
                        wait_write(b2)
                    fire_gathers(s + 1, b2)

                wait_gathers(b)
                fire_write(s, b)

        wait_write(0)
        wait_write(1)

    return body(table, ids)


def kernel(token_ids, Embedding_Matrix):
    return _gather_sc(Embedding_Matrix, token_ids.astype(jnp.int32))
